# baseline (speedup 1.0000x reference)
import functools

import jax
import jax.numpy as jnp
from jax import lax
from jax.experimental import pallas as pl
from jax.experimental.pallas import tpu as pltpu
from jax.experimental.pallas import tpu_sc as plsc

TOP_P = 0.5
EPS = 1e-10
L = 512
E = 3
N = 64
P = 1
NPH = N // P
NBLK = 8
STEPS = NPH // NBLK
LCHUNK = L // STEPS
NC = 2
NS = 16
NW = NC * NS
_RPT = (N * L) // (P * NW)
CHUNKS = [80] * (_RPT // 80) + ([_RPT % 80] if _RPT % 80 else [])
OFFS = [sum(CHUNKS[:i]) for i in range(len(CHUNKS))]
NBUF = 3
CHMAX = max(CHUNKS)


def _gate_math(x_ref, wg_ref):
    xb = x_ref[...].reshape(NBLK * L, L)
    wg = wg_ref[...]
    zT = lax.dot_general(wg, xb, (((1,), (1,)), ((), ())),
                         preferred_element_type=jnp.float32)
    zmax = jnp.max(zT, axis=0, keepdims=True)
    ez = jnp.exp(zT - zmax)
    p = ez / jnp.sum(ez, axis=0, keepdims=True)
    ent = -jnp.sum(p * jnp.log(p + EPS))

    p0, p1, p2 = p[0:1], p[1:2], p[2:3]
    cb0 = p1 * (p1 > p0) + p2 * (p2 > p0)
    cb1 = p0 * (p0 >= p1) + p2 * (p2 > p1)
    cb2 = p0 * (p0 >= p2) + p1 * (p1 >= p2)
    g0 = (cb0 <= TOP_P).astype(jnp.int32)
    g1 = (cb1 <= TOP_P).astype(jnp.int32)
    g2 = (cb2 <= TOP_P).astype(jnp.int32)
    combo = g0 + 2 * g1 + 4 * g2
    lane = lax.broadcasted_iota(jnp.int32, (1, NBLK * L), 1)
    ridx = combo * L + (lane & (L - 1))

    mx = jnp.maximum(jnp.maximum(p0, p1), p2)
    mn = jnp.minimum(jnp.minimum(p0, p1), p2)
    med = jnp.maximum(jnp.minimum(p0, p1), jnp.minimum(jnp.maximum(p0, p1), p2))
    k1 = (mx <= TOP_P).astype(jnp.float32)
    k2 = ((mx + med) <= TOP_P).astype(jnp.float32)
    contrib = jnp.concatenate([mx, med * k1, mn * k2], axis=0)
    folded = contrib[:, 0:L]
    for b in range(1, NBLK):
        folded = folded + contrib[:, b * L:(b + 1) * L]
    return ridx, folded, ent


def _accumulate(n, folded, ent, sp_in_ref, ent_in_ref, sp_ref, ent_ref,
                loss_ref, last):
    @pl.when(n == 0)
    def _():
        sp_ref[...] = sp_in_ref[...] + folded
        ent_ref[0, 0] = ent_in_ref[0, 0] + ent

    @pl.when(n != 0)
    def _():
        sp_ref[...] = sp_ref[...] + folded
        ent_ref[0, 0] = ent_ref[0, 0] + ent

    @pl.when(n == last)
    def _():
        sp = sp_ref[...]
        mean = jnp.sum(sp) / (L * E)
        var = jnp.sum((sp - mean) ** 2) / (L * E - 1)
        loss_imp = var / (mean * mean + EPS)
        loss_ref[0, 0] = loss_imp + 0.1 * (ent_ref[0, 0] / (N * E))


def _tc_body0(x_ref, wg_ref, masks_ref, sp_in_ref, ent_in_ref,
              var_ref, idx_ref, sp_ref, ent_ref, loss_ref):
    n = pl.program_id(0)
    ridx, folded, ent = _gate_math(x_ref, wg_ref)
    idx_ref[0] = ridx
    _accumulate(n, folded, ent, sp_in_ref, ent_in_ref, sp_ref, ent_ref,
                loss_ref, STEPS - 1)
    m0 = masks_ref[:, 0, :]
    m1 = masks_ref[:, 1, :]
    m2 = masks_ref[:, 2, :]
    row = n * LCHUNK + lax.broadcasted_iota(jnp.int32, (LCHUNK, L), 0)
    col = lax.broadcasted_iota(jnp.int32, (LCHUNK, L), 1)
    eye = (row == col).astype(jnp.float32)
    var_ref[0] = eye
    var_ref[1] = m0 + eye
    var_ref[2] = m1 + eye
    var_ref[3] = (m0 + m1) + eye
    var_ref[4] = m2 + eye
    var_ref[5] = (m0 + m2) + eye
    var_ref[6] = (m1 + m2) + eye
    var_ref[7] = ((m0 + m1) + m2) + eye


def _tc_bodyN(x_ref, wg_ref, sp_in_ref, ent_in_ref,
              idx_ref, sp_ref, ent_ref, loss_ref):
    n = pl.program_id(0)
    ridx, folded, ent = _gate_math(x_ref, wg_ref)
    idx_ref[0] = ridx
    _accumulate(n, folded, ent, sp_in_ref, ent_in_ref, sp_ref, ent_ref,
                loss_ref, STEPS - 1)


def _tc_phase(phase, x_flat, W_gate, masks_t, sp_in, ent_in):
    common_out_specs = [
        pl.BlockSpec((1, 1, NBLK * L), lambda n: (n, 0, 0)),
        pl.BlockSpec((E, L), lambda n: (0, 0)),
        pl.BlockSpec(memory_space=pltpu.SMEM, block_shape=(1, 1),
                     index_map=lambda n: (0, 0)),
        pl.BlockSpec(memory_space=pltpu.SMEM, block_shape=(1, 1),
                     index_map=lambda n: (0, 0)),
    ]
    common_out_shape = [
        jax.ShapeDtypeStruct((STEPS, 1, NBLK * L), jnp.int32),
        jax.ShapeDtypeStruct((E, L), jnp.float32),
        jax.ShapeDtypeStruct((1, 1), jnp.float32),
        jax.ShapeDtypeStruct((1, 1), jnp.float32),
    ]
    x_spec = pl.BlockSpec((NBLK, L, L),
                          lambda n, p=phase: (p * STEPS + n, 0, 0))
    common_in_specs = [
        pl.BlockSpec((E, L), lambda n: (0, 0)),
        pl.BlockSpec((E, L), lambda n: (0, 0)),
        pl.BlockSpec(memory_space=pltpu.SMEM, block_shape=(1, 1),
                     index_map=lambda n: (0, 0)),
    ]
    if phase == 0:
        return pl.pallas_call(
            _tc_body0,
            grid=(STEPS,),
            in_specs=[x_spec,
                      common_in_specs[0],
                      pl.BlockSpec((LCHUNK, E, L), lambda n: (n, 0, 0)),
                      common_in_specs[1], common_in_specs[2]],
            out_specs=[pl.BlockSpec((8, LCHUNK, L), lambda n: (0, n, 0))]
            + common_out_specs,
            out_shape=[jax.ShapeDtypeStruct((8, L, L), jnp.float32)]
            + common_out_shape,
        )(x_flat, W_gate, masks_t, sp_in, ent_in)
    outs = pl.pallas_call(
        _tc_bodyN,
        grid=(STEPS,),
        in_specs=[x_spec] + common_in_specs,
        out_specs=common_out_specs,
        out_shape=common_out_shape,
    )(x_flat, W_gate, sp_in, ent_in)
    return (None,) + tuple(outs)


def _make_sc_body(phase):
    base = phase * NPH * L

    def _sc_body(var_hbm, idx_hbm, out_hbm, idx_v, stg0, stg1, stg2,
                 gs0, gs1, gs2, ws0, ws1, ws2):
        wid = lax.axis_index("s") * NC + lax.axis_index("c")
        rows_per = (NPH * L) // NW
        r0 = wid * rows_per
        pltpu.sync_copy(idx_hbm.at[pl.ds(r0, rows_per)], idx_v)
        stg = (stg0, stg1, stg2)
        gs = (gs0, gs1, gs2)
        ws = (ws0, ws1, ws2)
        gd = [None] * NBUF
        wd = [None] * NBUF
        parts = len(CHUNKS)
        for c in range(parts):
            s = c % NBUF
            if wd[s] is not None:
                wd[s].wait()
            gd[s] = pltpu.async_copy(
                var_hbm.at[idx_v.at[pl.ds(OFFS[c], CHUNKS[c])]],
                stg[s].at[pl.ds(0, CHUNKS[c])], gs[s])
            if c >= 1:
                sp = (c - 1) % NBUF
                gd[sp].wait()
                wd[sp] = pltpu.async_copy(
                    stg[sp].at[pl.ds(0, CHUNKS[c - 1])],
                    out_hbm.at[pl.ds(base + r0 + OFFS[c - 1], CHUNKS[c - 1])],
                    ws[sp])
        lastb = (parts - 1) % NBUF
        gd[lastb].wait()
        wd[lastb] = pltpu.async_copy(
            stg[lastb].at[pl.ds(0, CHUNKS[parts - 1])],
            out_hbm.at[pl.ds(base + r0 + OFFS[parts - 1], CHUNKS[parts - 1])],
            ws[lastb])
        for s in range(NBUF):
            if wd[s] is not None:
                wd[s].wait()

    return _sc_body


def _sc_phase(phase, variants_flat, ridx_flat):
    mesh = plsc.VectorSubcoreMesh(core_axis_name="c", subcore_axis_name="s")
    run = functools.partial(
        pl.kernel,
        out_type=jax.ShapeDtypeStruct((N * L, L), jnp.float32),
        mesh=mesh,
        scratch_types=[
            pltpu.VMEM(((NPH * L) // NW,), jnp.int32),
            pltpu.VMEM((CHMAX, L), jnp.float32),
            pltpu.VMEM((CHMAX, L), jnp.float32),
            pltpu.VMEM((CHMAX, L), jnp.float32),
            pltpu.SemaphoreType.DMA,
            pltpu.SemaphoreType.DMA,
            pltpu.SemaphoreType.DMA,
            pltpu.SemaphoreType.DMA,
            pltpu.SemaphoreType.DMA,
            pltpu.SemaphoreType.DMA,
        ],
    )(_make_sc_body(phase))
    return run(variants_flat, ridx_flat)


def kernel(x, masks, W_gate, W_noise):
    B, H, _, _ = x.shape
    x_flat = x.reshape(B * H, L, L)
    masks_t = masks
    sp = jnp.zeros((E, L), jnp.float32)
    ent = jnp.zeros((1, 1), jnp.float32)
    variants = None
    loss = None
    for p in range(P):
        var_p, ridx, sp, ent, loss = _tc_phase(p, x_flat, W_gate, masks_t,
                                               sp, ent)
        if p == 0:
            variants = var_p.reshape(8 * L, L)
        out = _sc_phase(p, variants, ridx.reshape(NPH * L))
    return out.reshape(B, H, L, L), loss[0, 0]


# speedup vs baseline: 1.0858x; 1.0858x over previous
import functools

import jax
import jax.numpy as jnp
from jax import lax
from jax.experimental import pallas as pl
from jax.experimental.pallas import tpu as pltpu
from jax.experimental.pallas import tpu_sc as plsc

TOP_P = 0.5
EPS = 1e-10
L = 512
E = 3
N = 64
P = 1
NPH = N // P
NBLK = 8
STEPS = NPH // NBLK
LCHUNK = L // STEPS
NC = 2
NS = 16
NW = NC * NS
_RPT = (N * L) // (P * NW)
CHUNKS = [80] * (_RPT // 80) + ([_RPT % 80] if _RPT % 80 else [])
OFFS = [sum(CHUNKS[:i]) for i in range(len(CHUNKS))]
NBUF = 3
CHMAX = max(CHUNKS)


def _gate_math(x_ref, wg_ref):
    xb = x_ref[...].reshape(NBLK * L, L)
    wg = wg_ref[...]
    zT = lax.dot_general(wg, xb, (((1,), (1,)), ((), ())),
                         preferred_element_type=jnp.float32)
    zmax = jnp.max(zT, axis=0, keepdims=True)
    ez = jnp.exp(zT - zmax)
    p = ez / jnp.sum(ez, axis=0, keepdims=True)
    ent = -jnp.sum(p * jnp.log(p + EPS))

    p0, p1, p2 = p[0:1], p[1:2], p[2:3]
    cb0 = p1 * (p1 > p0) + p2 * (p2 > p0)
    cb1 = p0 * (p0 >= p1) + p2 * (p2 > p1)
    cb2 = p0 * (p0 >= p2) + p1 * (p1 >= p2)
    g0 = (cb0 <= TOP_P).astype(jnp.int32)
    g1 = (cb1 <= TOP_P).astype(jnp.int32)
    g2 = (cb2 <= TOP_P).astype(jnp.int32)
    combo = g0 + 2 * g1 + 4 * g2
    lane = lax.broadcasted_iota(jnp.int32, (1, NBLK * L), 1)
    ridx = combo * L + (lane & (L - 1))

    mx = jnp.maximum(jnp.maximum(p0, p1), p2)
    mn = jnp.minimum(jnp.minimum(p0, p1), p2)
    med = jnp.maximum(jnp.minimum(p0, p1), jnp.minimum(jnp.maximum(p0, p1), p2))
    k1 = (mx <= TOP_P).astype(jnp.float32)
    k2 = ((mx + med) <= TOP_P).astype(jnp.float32)
    contrib = jnp.concatenate([mx, med * k1, mn * k2], axis=0)
    folded = contrib[:, 0:L]
    for b in range(1, NBLK):
        folded = folded + contrib[:, b * L:(b + 1) * L]
    return ridx, folded, ent


def _accumulate(n, folded, ent, sp_in_ref, ent_in_ref, sp_ref, ent_ref,
                loss_ref, last):
    @pl.when(n == 0)
    def _():
        sp_ref[...] = sp_in_ref[...] + folded
        ent_ref[0, 0] = ent_in_ref[0, 0] + ent

    @pl.when(n != 0)
    def _():
        sp_ref[...] = sp_ref[...] + folded
        ent_ref[0, 0] = ent_ref[0, 0] + ent

    @pl.when(n == last)
    def _():
        sp = sp_ref[...]
        mean = jnp.sum(sp) / (L * E)
        var = jnp.sum((sp - mean) ** 2) / (L * E - 1)
        loss_imp = var / (mean * mean + EPS)
        loss_ref[0, 0] = loss_imp + 0.1 * (ent_ref[0, 0] / (N * E))


def _tc_body0(x_ref, wg_ref, masks_ref, sp_in_ref, ent_in_ref,
              var_ref, idx_ref, sp_ref, ent_ref, loss_ref):
    n = pl.program_id(0)
    ridx, folded, ent = _gate_math(x_ref, wg_ref)
    idx_ref[0] = ridx
    _accumulate(n, folded, ent, sp_in_ref, ent_in_ref, sp_ref, ent_ref,
                loss_ref, STEPS - 1)
    m0 = masks_ref[0]
    m1 = masks_ref[1]
    m2 = masks_ref[2]
    row = n * LCHUNK + lax.broadcasted_iota(jnp.int32, (LCHUNK, L), 0)
    col = lax.broadcasted_iota(jnp.int32, (LCHUNK, L), 1)
    eye = (row == col).astype(jnp.float32)
    var_ref[0] = eye
    var_ref[1] = m0 + eye
    var_ref[2] = m1 + eye
    var_ref[3] = (m0 + m1) + eye
    var_ref[4] = m2 + eye
    var_ref[5] = (m0 + m2) + eye
    var_ref[6] = (m1 + m2) + eye
    var_ref[7] = ((m0 + m1) + m2) + eye


def _tc_bodyN(x_ref, wg_ref, sp_in_ref, ent_in_ref,
              idx_ref, sp_ref, ent_ref, loss_ref):
    n = pl.program_id(0)
    ridx, folded, ent = _gate_math(x_ref, wg_ref)
    idx_ref[0] = ridx
    _accumulate(n, folded, ent, sp_in_ref, ent_in_ref, sp_ref, ent_ref,
                loss_ref, STEPS - 1)


def _tc_phase(phase, x_flat, W_gate, masks_t, sp_in, ent_in):
    common_out_specs = [
        pl.BlockSpec((1, 1, NBLK * L), lambda n: (n, 0, 0)),
        pl.BlockSpec((E, L), lambda n: (0, 0)),
        pl.BlockSpec(memory_space=pltpu.SMEM, block_shape=(1, 1),
                     index_map=lambda n: (0, 0)),
        pl.BlockSpec(memory_space=pltpu.SMEM, block_shape=(1, 1),
                     index_map=lambda n: (0, 0)),
    ]
    common_out_shape = [
        jax.ShapeDtypeStruct((STEPS, 1, NBLK * L), jnp.int32),
        jax.ShapeDtypeStruct((E, L), jnp.float32),
        jax.ShapeDtypeStruct((1, 1), jnp.float32),
        jax.ShapeDtypeStruct((1, 1), jnp.float32),
    ]
    x_spec = pl.BlockSpec((NBLK, L, L),
                          lambda n, p=phase: (p * STEPS + n, 0, 0))
    common_in_specs = [
        pl.BlockSpec((E, L), lambda n: (0, 0)),
        pl.BlockSpec((E, L), lambda n: (0, 0)),
        pl.BlockSpec(memory_space=pltpu.SMEM, block_shape=(1, 1),
                     index_map=lambda n: (0, 0)),
    ]
    if phase == 0:
        return pl.pallas_call(
            _tc_body0,
            grid=(STEPS,),
            in_specs=[x_spec,
                      common_in_specs[0],
                      pl.BlockSpec((E, LCHUNK, L), lambda n: (0, n, 0)),
                      common_in_specs[1], common_in_specs[2]],
            out_specs=[pl.BlockSpec((8, LCHUNK, L), lambda n: (0, n, 0))]
            + common_out_specs,
            out_shape=[jax.ShapeDtypeStruct((8, L, L), jnp.float32)]
            + common_out_shape,
        )(x_flat, W_gate, masks_t, sp_in, ent_in)
    outs = pl.pallas_call(
        _tc_bodyN,
        grid=(STEPS,),
        in_specs=[x_spec] + common_in_specs,
        out_specs=common_out_specs,
        out_shape=common_out_shape,
    )(x_flat, W_gate, sp_in, ent_in)
    return (None,) + tuple(outs)


def _make_sc_body(phase):
    base = phase * NPH * L

    def _sc_body(var_hbm, idx_hbm, out_hbm, idx_v, stg0, stg1, stg2,
                 gs0, gs1, gs2, ws0, ws1, ws2):
        wid = lax.axis_index("s") * NC + lax.axis_index("c")
        rows_per = (NPH * L) // NW
        r0 = wid * rows_per
        pltpu.sync_copy(idx_hbm.at[pl.ds(r0, rows_per)], idx_v)
        stg = (stg0, stg1, stg2)
        gs = (gs0, gs1, gs2)
        ws = (ws0, ws1, ws2)
        gd = [None] * NBUF
        wd = [None] * NBUF
        parts = len(CHUNKS)
        for c in range(parts):
            s = c % NBUF
            if wd[s] is not None:
                wd[s].wait()
            gd[s] = pltpu.async_copy(
                var_hbm.at[idx_v.at[pl.ds(OFFS[c], CHUNKS[c])]],
                stg[s].at[pl.ds(0, CHUNKS[c])], gs[s])
            if c >= 1:
                sp = (c - 1) % NBUF
                gd[sp].wait()
                wd[sp] = pltpu.async_copy(
                    stg[sp].at[pl.ds(0, CHUNKS[c - 1])],
                    out_hbm.at[pl.ds(base + r0 + OFFS[c - 1], CHUNKS[c - 1])],
                    ws[sp])
        lastb = (parts - 1) % NBUF
        gd[lastb].wait()
        wd[lastb] = pltpu.async_copy(
            stg[lastb].at[pl.ds(0, CHUNKS[parts - 1])],
            out_hbm.at[pl.ds(base + r0 + OFFS[parts - 1], CHUNKS[parts - 1])],
            ws[lastb])
        for s in range(NBUF):
            if wd[s] is not None:
                wd[s].wait()

    return _sc_body


def _sc_phase(phase, variants_flat, ridx_flat):
    mesh = plsc.VectorSubcoreMesh(core_axis_name="c", subcore_axis_name="s")
    run = functools.partial(
        pl.kernel,
        out_type=jax.ShapeDtypeStruct((N * L, L), jnp.float32),
        mesh=mesh,
        scratch_types=[
            pltpu.VMEM(((NPH * L) // NW,), jnp.int32),
            pltpu.VMEM((CHMAX, L), jnp.float32),
            pltpu.VMEM((CHMAX, L), jnp.float32),
            pltpu.VMEM((CHMAX, L), jnp.float32),
            pltpu.SemaphoreType.DMA,
            pltpu.SemaphoreType.DMA,
            pltpu.SemaphoreType.DMA,
            pltpu.SemaphoreType.DMA,
            pltpu.SemaphoreType.DMA,
            pltpu.SemaphoreType.DMA,
        ],
    )(_make_sc_body(phase))
    return run(variants_flat, ridx_flat)


def kernel(x, masks, W_gate, W_noise):
    B, H, _, _ = x.shape
    x_flat = x.reshape(B * H, L, L)
    masks_t = jnp.transpose(masks, (1, 0, 2))
    sp = jnp.zeros((E, L), jnp.float32)
    ent = jnp.zeros((1, 1), jnp.float32)
    variants = None
    loss = None
    for p in range(P):
        var_p, ridx, sp, ent, loss = _tc_phase(p, x_flat, W_gate, masks_t,
                                               sp, ent)
        if p == 0:
            variants = var_p.reshape(8 * L, L)
        out = _sc_phase(p, variants, ridx.reshape(NPH * L))
    return out.reshape(B, H, L, L), loss[0, 0]


# speedup vs baseline: 1.1006x; 1.0136x over previous
import functools

import jax
import jax.numpy as jnp
from jax import lax
from jax.experimental import pallas as pl
from jax.experimental.pallas import tpu as pltpu
from jax.experimental.pallas import tpu_sc as plsc

TOP_P = 0.5
EPS = 1e-10
L = 512
E = 3
N = 64
NBLK = 8
STEPS = N // NBLK
LCHUNK = L // STEPS
NC = 2
NS = 16
NW = NC * NS
RPT = (N * L) // NW
CHUNKS = [80] * (RPT // 80) + ([RPT % 80] if RPT % 80 else [])
OFFS = [sum(CHUNKS[:i]) for i in range(len(CHUNKS))]
NBUF = 3
CHMAX = max(CHUNKS)


def _tc_body(x_ref, wg_ref, masks_ref, var_ref, idx_ref, sp_ref, ent_ref,
             loss_ref):
    n = pl.program_id(0)
    xb = x_ref[...].reshape(NBLK * L, L)
    wg = wg_ref[...]
    zT = lax.dot_general(wg, xb, (((1,), (1,)), ((), ())),
                         preferred_element_type=jnp.float32)
    zmax = jnp.max(zT, axis=0, keepdims=True)
    ez = jnp.exp(zT - zmax)
    p = ez / jnp.sum(ez, axis=0, keepdims=True)
    ent = -jnp.sum(p * jnp.log(p + EPS))

    p0, p1, p2 = p[0:1], p[1:2], p[2:3]
    cb0 = p1 * (p1 > p0) + p2 * (p2 > p0)
    cb1 = p0 * (p0 >= p1) + p2 * (p2 > p1)
    cb2 = p0 * (p0 >= p2) + p1 * (p1 >= p2)
    g0 = (cb0 <= TOP_P).astype(jnp.int32)
    g1 = (cb1 <= TOP_P).astype(jnp.int32)
    g2 = (cb2 <= TOP_P).astype(jnp.int32)
    combo = g0 + 2 * g1 + 4 * g2
    lane = lax.broadcasted_iota(jnp.int32, (1, NBLK * L), 1)
    idx_ref[0] = combo * L + (lane & (L - 1))

    mx = jnp.maximum(jnp.maximum(p0, p1), p2)
    mn = jnp.minimum(jnp.minimum(p0, p1), p2)
    med = jnp.maximum(jnp.minimum(p0, p1), jnp.minimum(jnp.maximum(p0, p1), p2))
    k1 = (mx <= TOP_P).astype(jnp.float32)
    k2 = ((mx + med) <= TOP_P).astype(jnp.float32)
    contrib = jnp.concatenate([mx, med * k1, mn * k2], axis=0)
    folded = contrib[:, 0:L]
    for b in range(1, NBLK):
        folded = folded + contrib[:, b * L:(b + 1) * L]

    @pl.when(n == 0)
    def _():
        sp_ref[...] = folded
        ent_ref[0, 0] = ent

    @pl.when(n != 0)
    def _():
        sp_ref[...] = sp_ref[...] + folded
        ent_ref[0, 0] = ent_ref[0, 0] + ent

    m0 = masks_ref[0]
    m1 = masks_ref[1]
    m2 = masks_ref[2]
    row = n * LCHUNK + lax.broadcasted_iota(jnp.int32, (LCHUNK, L), 0)
    col = lax.broadcasted_iota(jnp.int32, (LCHUNK, L), 1)
    eye = (row == col).astype(jnp.float32)
    var_ref[0] = eye
    var_ref[1] = m0 + eye
    var_ref[2] = m1 + eye
    var_ref[3] = (m0 + m1) + eye
    var_ref[4] = m2 + eye
    var_ref[5] = (m0 + m2) + eye
    var_ref[6] = (m1 + m2) + eye
    var_ref[7] = ((m0 + m1) + m2) + eye

    @pl.when(n == STEPS - 1)
    def _():
        sp = sp_ref[...]
        mean = jnp.sum(sp) / (L * E)
        var = jnp.sum((sp - mean) ** 2) / (L * E - 1)
        loss_imp = var / (mean * mean + EPS)
        loss_ref[0, 0] = loss_imp + 0.1 * (ent_ref[0, 0] / (N * E))


def _tc_stage(x_flat, masks_t, W_gate):
    smem_scalar = functools.partial(
        pl.BlockSpec, memory_space=pltpu.SMEM, block_shape=(1, 1),
        index_map=lambda n: (0, 0))
    return pl.pallas_call(
        _tc_body,
        grid=(STEPS,),
        in_specs=[
            pl.BlockSpec((NBLK, L, L), lambda n: (n, 0, 0)),
            pl.BlockSpec((E, L), lambda n: (0, 0)),
            pl.BlockSpec((E, LCHUNK, L), lambda n: (0, n, 0)),
        ],
        out_specs=[
            pl.BlockSpec((8, LCHUNK, L), lambda n: (0, n, 0)),
            pl.BlockSpec((1, 1, NBLK * L), lambda n: (n, 0, 0)),
            pl.BlockSpec((E, L), lambda n: (0, 0)),
            smem_scalar(),
            smem_scalar(),
        ],
        out_shape=[
            jax.ShapeDtypeStruct((8, L, L), jnp.float32),
            jax.ShapeDtypeStruct((STEPS, 1, NBLK * L), jnp.int32),
            jax.ShapeDtypeStruct((E, L), jnp.float32),
            jax.ShapeDtypeStruct((1, 1), jnp.float32),
            jax.ShapeDtypeStruct((1, 1), jnp.float32),
        ],
    )(x_flat, W_gate, masks_t)


def _sc_body(var_hbm, idx_hbm, out_hbm, idx_v, stg0, stg1, stg2,
             gs0, gs1, gs2, ws0, ws1, ws2):
    wid = lax.axis_index("s") * NC + lax.axis_index("c")
    r0 = wid * RPT
    pltpu.sync_copy(idx_hbm.at[pl.ds(r0, RPT)], idx_v)
    stg = (stg0, stg1, stg2)
    gs = (gs0, gs1, gs2)
    ws = (ws0, ws1, ws2)
    gd = [None] * NBUF
    wd = [None] * NBUF
    parts = len(CHUNKS)
    for c in range(parts):
        s = c % NBUF
        if wd[s] is not None:
            wd[s].wait()
        gd[s] = pltpu.async_copy(
            var_hbm.at[idx_v.at[pl.ds(OFFS[c], CHUNKS[c])]],
            stg[s].at[pl.ds(0, CHUNKS[c])], gs[s])
        if c >= 1:
            sp = (c - 1) % NBUF
            gd[sp].wait()
            wd[sp] = pltpu.async_copy(
                stg[sp].at[pl.ds(0, CHUNKS[c - 1])],
                out_hbm.at[pl.ds(r0 + OFFS[c - 1], CHUNKS[c - 1])], ws[sp])
    lastb = (parts - 1) % NBUF
    gd[lastb].wait()
    wd[lastb] = pltpu.async_copy(
        stg[lastb].at[pl.ds(0, CHUNKS[parts - 1])],
        out_hbm.at[pl.ds(r0 + OFFS[parts - 1], CHUNKS[parts - 1])], ws[lastb])
    for s in range(NBUF):
        if wd[s] is not None:
            wd[s].wait()


def _sc_stage(variants_flat, ridx_flat):
    mesh = plsc.VectorSubcoreMesh(core_axis_name="c", subcore_axis_name="s")
    run = functools.partial(
        pl.kernel,
        out_type=jax.ShapeDtypeStruct((N * L, L), jnp.float32),
        mesh=mesh,
        scratch_types=[
            pltpu.VMEM((RPT,), jnp.int32),
            pltpu.VMEM((CHMAX, L), jnp.float32),
            pltpu.VMEM((CHMAX, L), jnp.float32),
            pltpu.VMEM((CHMAX, L), jnp.float32),
            pltpu.SemaphoreType.DMA,
            pltpu.SemaphoreType.DMA,
            pltpu.SemaphoreType.DMA,
            pltpu.SemaphoreType.DMA,
            pltpu.SemaphoreType.DMA,
            pltpu.SemaphoreType.DMA,
        ],
    )(_sc_body)
    return run(variants_flat, ridx_flat)


def kernel(x, masks, W_gate, W_noise):
    B, H, _, _ = x.shape
    x_flat = x.reshape(B * H, L, L)
    masks_t = jnp.transpose(masks, (1, 0, 2))
    variants, ridx, _sp, _ent, loss = _tc_stage(x_flat, masks_t, W_gate)
    out = _sc_stage(variants.reshape(8 * L, L), ridx.reshape(N * L))
    return out.reshape(B, H, L, L), loss[0, 0]
